# baseline (speedup 1.0000x reference)
import functools

import jax
import jax.numpy as jnp
from jax import lax
from jax.experimental import pallas as pl
from jax.experimental.pallas import tpu as pltpu, tpu_sc as plsc

BOARD = 19
NPOS = BOARD * BOARD
D = 128
M = 16384
NPAD = 384
NC, NS, L = 2, 16, 16
NW = NC * NS
ROWS_PER_W = M // NW
CHUNK = 128
N_CHUNKS = ROWS_PER_W // CHUNK
IDX_BM = 2048


def _table_body(p_ref, wt_ref, b_ref, o_ref):
    i = pl.program_id(0)
    acc = jnp.dot(p_ref[...], wt_ref[...], preferred_element_type=jnp.float32)
    bias = jnp.where(i == 0, b_ref[...], jnp.zeros_like(b_ref[...]))
    o = (acc + bias).astype(jnp.bfloat16)
    o_ref[...] = jnp.concatenate([o, jnp.zeros_like(o)], axis=1)


def _build_table(p_pad, wt, b_row):
    return pl.pallas_call(
        _table_body,
        grid=(2,),
        in_specs=[
            pl.BlockSpec((NPAD, D), lambda i: (0, 0)),
            pl.BlockSpec((D, D), lambda i: (i, 0)),
            pl.BlockSpec((1, D), lambda i: (0, 0)),
        ],
        out_specs=pl.BlockSpec((NPAD, 2 * D), lambda i: (i, 0)),
        out_shape=jax.ShapeDtypeStruct((2 * NPAD, 2 * D), jnp.bfloat16),
    )(p_pad, wt, b_row)


def _idx_body(fxy_ref, txy_ref, fi_ref, ti_ref):
    col = lax.broadcasted_iota(jnp.int32, (1, 2), 1)
    coef = jnp.where(col == 0, BOARD, 1)
    fi = jnp.sum(fxy_ref[...] * coef, axis=1)
    ti = jnp.sum(txy_ref[...] * coef, axis=1)
    fi_ref[...] = jnp.clip(fi, 0, NPOS - 1)
    ti_ref[...] = jnp.clip(ti, 0, NPOS - 1) + NPAD


def _build_indices(fxy, txy):
    return pl.pallas_call(
        _idx_body,
        grid=(M // IDX_BM,),
        in_specs=[
            pl.BlockSpec((IDX_BM, 2), lambda i: (i, 0)),
            pl.BlockSpec((IDX_BM, 2), lambda i: (i, 0)),
        ],
        out_specs=[
            pl.BlockSpec((IDX_BM,), lambda i: (i,)),
            pl.BlockSpec((IDX_BM,), lambda i: (i,)),
        ],
        out_shape=[
            jax.ShapeDtypeStruct((M,), jnp.int32),
            jax.ShapeDtypeStruct((M,), jnp.int32),
        ],
    )(fxy, txy)


def _sc_body(fx_hbm, fy_hbm, tx_hbm, ty_hbm, tbl_hbm, out_hbm,
             fx_v, fy_v, tx_v, ty_v, fi_v, ti_v, rows_a, rows_b, o_v,
             sem_a0, sem_a1, sem_b0, sem_b1, sem_w0, sem_w1, sem_c0, sem_c1):
    sem_a = (sem_a0, sem_a1)
    sem_b = (sem_b0, sem_b1)
    sem_w = (sem_w0, sem_w1)
    sem_c = (sem_c0, sem_c1)
    wid = lax.axis_index("s") * NC + lax.axis_index("c")

    def coords_start(c):
        s = c % 2
        base = wid * ROWS_PER_W + c * CHUNK
        return [
            pltpu.async_copy(src.at[pl.ds(base, CHUNK)], dst.at[s], sem_c[s])
            for src, dst in ((fx_hbm, fx_v), (fy_hbm, fy_v),
                             (tx_hbm, tx_v), (ty_hbm, ty_v))
        ]

    def gathers_start(c, coord_cps):
        s = c % 2
        for cp in coord_cps:
            cp.wait()
        for j in range(CHUNK // L):
            sl = pl.ds(j * L, L)
            fi_v[s, sl] = jnp.clip(
                fx_v[s, sl] * BOARD + fy_v[s, sl], 0, NPOS - 1)
            ti_v[s, sl] = jnp.clip(
                tx_v[s, sl] * BOARD + ty_v[s, sl], 0, NPOS - 1) + NPAD
        cp_a = pltpu.async_copy(tbl_hbm.at[fi_v.at[s]], rows_a.at[s], sem_a[s])
        cp_b = pltpu.async_copy(tbl_hbm.at[ti_v.at[s]], rows_b.at[s], sem_b[s])
        return cp_a, cp_b

    coord_cps = [None] * N_CHUNKS
    gather_cps = [None] * N_CHUNKS
    writebacks = [None, None]

    coord_cps[0] = coords_start(0)
    if N_CHUNKS > 1:
        coord_cps[1] = coords_start(1)
    gather_cps[0] = gathers_start(0, coord_cps[0])

    for c in range(N_CHUNKS):
        s = c % 2
        if c + 2 < N_CHUNKS:
            coord_cps[c + 2] = coords_start(c + 2)
        if c + 1 < N_CHUNKS:
            ns = (c + 1) % 2
            if writebacks[ns] is not None:
                writebacks[ns].wait()
                writebacks[ns] = None
            gather_cps[c + 1] = gathers_start(c + 1, coord_cps[c + 1])
        gather_cps[c][0].wait()
        gather_cps[c][1].wait()

        def relu_body(r, _):
            for k in range(D // (2 * L)):
                sl32 = pl.ds(k * L, L)
                va = plsc.bitcast(rows_a[s, r, sl32], jnp.bfloat16)
                vb = plsc.bitcast(rows_b[s, r, sl32], jnp.bfloat16)
                v = jnp.maximum(va + vb, 0.0)
                even, odd = plsc.unpack(
                    v, format=plsc.PackFormat.INTERLEAVED,
                    preferred_element_type=jnp.float32)
                o_v[s, r, pl.ds(k * L, L)] = even
                o_v[s, r, pl.ds(D // 2 + k * L, L)] = odd
            return 0

        lax.fori_loop(0, CHUNK, relu_body, 0, unroll=4)

        base = wid * ROWS_PER_W + c * CHUNK
        writebacks[s] = pltpu.async_copy(
            o_v.at[s], out_hbm.at[pl.ds(base, CHUNK)], sem_w[s])
    for wb in writebacks:
        if wb is not None:
            wb.wait()


_sc_lookup = functools.partial(
    pl.kernel,
    out_type=jax.ShapeDtypeStruct((M, D), jnp.float32),
    mesh=plsc.VectorSubcoreMesh(
        core_axis_name="c", subcore_axis_name="s", num_cores=NC, num_subcores=NS
    ),
    compiler_params=pltpu.CompilerParams(needs_layout_passes=False),
    scratch_types=[
        pltpu.VMEM((2, CHUNK), jnp.int32),
        pltpu.VMEM((2, CHUNK), jnp.int32),
        pltpu.VMEM((2, CHUNK), jnp.int32),
        pltpu.VMEM((2, CHUNK), jnp.int32),
        pltpu.VMEM((2, CHUNK), jnp.int32),
        pltpu.VMEM((2, CHUNK), jnp.int32),
        pltpu.VMEM((2, CHUNK, D), jnp.int32),
        pltpu.VMEM((2, CHUNK, D), jnp.int32),
        pltpu.VMEM((2, CHUNK, D), jnp.float32),
        pltpu.SemaphoreType.DMA,
        pltpu.SemaphoreType.DMA,
        pltpu.SemaphoreType.DMA,
        pltpu.SemaphoreType.DMA,
        pltpu.SemaphoreType.DMA,
        pltpu.SemaphoreType.DMA,
        pltpu.SemaphoreType.DMA,
        pltpu.SemaphoreType.DMA,
    ],
)(_sc_body)


_PERM = [(j // 2) if j % 2 == 0 else (D // 2 + j // 2) for j in range(D)]


def kernel(from_xy, to_xy, pos_embed, W, b):
    p_pad = jnp.pad(pos_embed, ((0, NPAD - NPOS), (0, 0)))
    perm = jnp.array(_PERM, dtype=jnp.int32)
    wt = W.T[:, perm]
    b_row = b[perm].reshape(1, D)
    table = _build_table(p_pad, wt, b_row)
    table = jax.lax.bitcast_convert_type(
        table.reshape(2 * NPAD, D, 2), jnp.int32)
    fxy = from_xy.astype(jnp.int32)
    txy = to_xy.astype(jnp.int32)
    return _sc_lookup(fxy[:, 0], fxy[:, 1], txy[:, 0], txy[:, 1], table)


# speedup vs baseline: 1.1707x; 1.1707x over previous
import functools

import jax
import jax.numpy as jnp
from jax import lax
from jax.experimental import pallas as pl
from jax.experimental.pallas import tpu as pltpu, tpu_sc as plsc

BOARD = 19
NPOS = BOARD * BOARD
D = 128
M = 16384
NPAD = 384
NC, NS, L = 2, 16, 16
NW = NC * NS
ROWS_PER_W = M // NW
CHUNK = 128
N_CHUNKS = ROWS_PER_W // CHUNK
IDX_BM = 2048


def _table_body(p_ref, wt_ref, b_ref, o_ref):
    i = pl.program_id(0)
    acc = jnp.dot(p_ref[...], wt_ref[...], preferred_element_type=jnp.float32)
    bias = jnp.where(i == 0, b_ref[...], jnp.zeros_like(b_ref[...]))
    o_ref[...] = acc + bias


def _build_table(p_pad, wt, b_row):
    return pl.pallas_call(
        _table_body,
        grid=(2,),
        in_specs=[
            pl.BlockSpec((NPAD, D), lambda i: (0, 0)),
            pl.BlockSpec((D, D), lambda i: (i, 0)),
            pl.BlockSpec((1, D), lambda i: (0, 0)),
        ],
        out_specs=pl.BlockSpec((NPAD, D), lambda i: (i, 0)),
        out_shape=jax.ShapeDtypeStruct((2 * NPAD, D), jnp.float32),
    )(p_pad, wt, b_row)


def _idx_body(fxy_ref, txy_ref, fi_ref, ti_ref):
    col = lax.broadcasted_iota(jnp.int32, (1, 2), 1)
    coef = jnp.where(col == 0, BOARD, 1)
    fi = jnp.sum(fxy_ref[...] * coef, axis=1)
    ti = jnp.sum(txy_ref[...] * coef, axis=1)
    fi_ref[...] = jnp.clip(fi, 0, NPOS - 1)
    ti_ref[...] = jnp.clip(ti, 0, NPOS - 1) + NPAD


def _build_indices(fxy, txy):
    return pl.pallas_call(
        _idx_body,
        grid=(M // IDX_BM,),
        in_specs=[
            pl.BlockSpec((IDX_BM, 2), lambda i: (i, 0)),
            pl.BlockSpec((IDX_BM, 2), lambda i: (i, 0)),
        ],
        out_specs=[
            pl.BlockSpec((IDX_BM,), lambda i: (i,)),
            pl.BlockSpec((IDX_BM,), lambda i: (i,)),
        ],
        out_shape=[
            jax.ShapeDtypeStruct((M,), jnp.int32),
            jax.ShapeDtypeStruct((M,), jnp.int32),
        ],
    )(fxy, txy)


def _sc_body(fx_hbm, fy_hbm, tx_hbm, ty_hbm, tbl_hbm, out_hbm,
             fx_v, fy_v, tx_v, ty_v, fi_v, ti_v, rows_a, rows_b,
             sem_a0, sem_a1, sem_b0, sem_b1, sem_w0, sem_w1, sem_c0, sem_c1):
    sem_a = (sem_a0, sem_a1)
    sem_b = (sem_b0, sem_b1)
    sem_w = (sem_w0, sem_w1)
    sem_c = (sem_c0, sem_c1)
    wid = lax.axis_index("s") * NC + lax.axis_index("c")

    def coords_start(c):
        s = c % 2
        base = wid * ROWS_PER_W + c * CHUNK
        return [
            pltpu.async_copy(src.at[pl.ds(base, CHUNK)], dst.at[s], sem_c[s])
            for src, dst in ((fx_hbm, fx_v), (fy_hbm, fy_v),
                             (tx_hbm, tx_v), (ty_hbm, ty_v))
        ]

    def gathers_start(c, coord_cps):
        s = c % 2
        for cp in coord_cps:
            cp.wait()
        for j in range(CHUNK // L):
            sl = pl.ds(j * L, L)
            fi_v[s, sl] = jnp.clip(
                fx_v[s, sl] * BOARD + fy_v[s, sl], 0, NPOS - 1)
            ti_v[s, sl] = jnp.clip(
                tx_v[s, sl] * BOARD + ty_v[s, sl], 0, NPOS - 1) + NPAD
        cp_a = pltpu.async_copy(tbl_hbm.at[fi_v.at[s]], rows_a.at[s], sem_a[s])
        cp_b = pltpu.async_copy(tbl_hbm.at[ti_v.at[s]], rows_b.at[s], sem_b[s])
        return cp_a, cp_b

    coord_cps = [None] * N_CHUNKS
    gather_cps = [None] * N_CHUNKS
    writebacks = [None, None]

    coord_cps[0] = coords_start(0)
    if N_CHUNKS > 1:
        coord_cps[1] = coords_start(1)
    gather_cps[0] = gathers_start(0, coord_cps[0])

    for c in range(N_CHUNKS):
        s = c % 2
        if c + 2 < N_CHUNKS:
            coord_cps[c + 2] = coords_start(c + 2)
        if c + 1 < N_CHUNKS:
            ns = (c + 1) % 2
            if writebacks[ns] is not None:
                writebacks[ns].wait()
                writebacks[ns] = None
            gather_cps[c + 1] = gathers_start(c + 1, coord_cps[c + 1])
        gather_cps[c][0].wait()
        gather_cps[c][1].wait()

        @plsc.parallel_loop(0, CHUNK, step=1, unroll=4)
        def _relu_loop(r):
            for k in range(D // L):
                sl = pl.ds(k * L, L)
                a = rows_a[s, r, sl]
                bb = rows_b[s, r, sl]
                rows_a[s, r, sl] = jnp.maximum(a + bb, 0.0)

        base = wid * ROWS_PER_W + c * CHUNK
        writebacks[s] = pltpu.async_copy(
            rows_a.at[s], out_hbm.at[pl.ds(base, CHUNK)], sem_w[s])
    for wb in writebacks:
        if wb is not None:
            wb.wait()


_sc_lookup = functools.partial(
    pl.kernel,
    out_type=jax.ShapeDtypeStruct((M, D), jnp.float32),
    mesh=plsc.VectorSubcoreMesh(
        core_axis_name="c", subcore_axis_name="s", num_cores=NC, num_subcores=NS
    ),
    compiler_params=pltpu.CompilerParams(needs_layout_passes=False),
    scratch_types=[
        pltpu.VMEM((2, CHUNK), jnp.int32),
        pltpu.VMEM((2, CHUNK), jnp.int32),
        pltpu.VMEM((2, CHUNK), jnp.int32),
        pltpu.VMEM((2, CHUNK), jnp.int32),
        pltpu.VMEM((2, CHUNK), jnp.int32),
        pltpu.VMEM((2, CHUNK), jnp.int32),
        pltpu.VMEM((2, CHUNK, D), jnp.float32),
        pltpu.VMEM((2, CHUNK, D), jnp.float32),
        pltpu.SemaphoreType.DMA,
        pltpu.SemaphoreType.DMA,
        pltpu.SemaphoreType.DMA,
        pltpu.SemaphoreType.DMA,
        pltpu.SemaphoreType.DMA,
        pltpu.SemaphoreType.DMA,
        pltpu.SemaphoreType.DMA,
        pltpu.SemaphoreType.DMA,
    ],
)(_sc_body)


def kernel(from_xy, to_xy, pos_embed, W, b):
    p_pad = jnp.pad(pos_embed, ((0, NPAD - NPOS), (0, 0)))
    wt = W.T
    b_row = b.reshape(1, D)
    table = _build_table(p_pad, wt, b_row)
    fxy = from_xy.astype(jnp.int32)
    txy = to_xy.astype(jnp.int32)
    return _sc_lookup(fxy[:, 0], fxy[:, 1], txy[:, 0], txy[:, 1], table)
